# baseline (speedup 1.0000x reference)
import jax
import jax.numpy as jnp
from jax import lax
from jax.experimental import pallas as pl
from jax.experimental.pallas import tpu as pltpu
from jax.experimental.pallas import tpu_sc as plsc

_B = 16384
_D = 128
_DO = 2 * _D
_NC = 2
_NS = 16
_NW = _NC * _NS
_BW = _B // _NW
_CH = 128
_NCHUNK = _BW // _CH
_L = 16


def _tile_body(info_hbm, sex_hbm, bm_hbm, out_hbm, info_v, off_v,
               sex_t, bm_t, rows_a, rows_b, osems):
    wid = lax.axis_index("s") * _NC + lax.axis_index("c")
    base = wid * _BW
    pltpu.sync_copy(info_hbm.at[pl.ds(base * 2, _BW * 2)], info_v)
    pltpu.sync_copy(sex_hbm, sex_t)
    pltpu.sync_copy(bm_hbm, bm_t)
    lane2 = lax.iota(jnp.int32, _L) * 2
    for i in range(_BW // _L):
        s = plsc.load_gather(info_v, [lane2 + (2 * _L * i)])
        b = plsc.load_gather(info_v, [lane2 + (2 * _L * i + 1)])
        off_v[pl.ds(i * _L, _L)] = (s * _D) | ((b * _D) << 16)
    cps = [None, None]
    for j in range(_NCHUNK):
        buf = j % 2
        if cps[buf] is not None:
            cps[buf].wait()
        dst = (rows_a, rows_b)[buf]

        def group_body(g, _, j=j, dst=dst):
            ov = off_v[pl.ds(j * _CH + g * _L, _L)]
            offs = [ov[l] for l in range(_L)]
            for l in range(_L):
                so = offs[l] & 0xFFFF
                bo = lax.shift_right_logical(offs[l], 16)
                vs = ([sex_t[pl.ds(so + k, _L)] for k in range(0, _D, _L)]
                      + [bm_t[pl.ds(bo + k, _L)] for k in range(0, _D, _L)])
                for k, v in enumerate(vs):
                    dst[g * _L + l, pl.ds(k * _L, _L)] = v
            return 0

        lax.fori_loop(0, _CH // _L, group_body, 0)
        cps[buf] = pltpu.async_copy(
            dst,
            out_hbm.at[pl.ds(base + j * _CH, _CH)],
            osems.at[buf],
        )
    for cp in cps:
        cp.wait()


_lookup = pl.kernel(
    _tile_body,
    out_type=jax.ShapeDtypeStruct((_B, _DO), jnp.float32),
    mesh=plsc.VectorSubcoreMesh(core_axis_name="c", subcore_axis_name="s",
                                num_cores=_NC, num_subcores=_NS),
    compiler_params=pltpu.CompilerParams(needs_layout_passes=False),
    scratch_types=[
        pltpu.VMEM((_BW * 2,), jnp.int32),
        pltpu.VMEM((_BW,), jnp.int32),
        pltpu.VMEM((3 * _D,), jnp.float32),
        pltpu.VMEM((13 * _D,), jnp.float32),
        pltpu.VMEM((_CH, _DO), jnp.float32),
        pltpu.VMEM((_CH, _DO), jnp.float32),
        pltpu.SemaphoreType.DMA((2,)),
    ],
)


def kernel(child_info, emb_sex, emb_b_month):
    info = child_info.astype(jnp.int32).reshape(-1)
    return _lookup(info, emb_sex.reshape(-1), emb_b_month.reshape(-1))


# speedup vs baseline: 1.2084x; 1.2084x over previous
import jax
import jax.numpy as jnp
from jax import lax
from jax.experimental import pallas as pl
from jax.experimental.pallas import tpu as pltpu
from jax.experimental.pallas import tpu_sc as plsc

_B = 16384
_D = 128
_DO = 2 * _D
_NC = 2
_NS = 16
_NW = _NC * _NS
_BW = _B // _NW
_CH = 128
_NCHUNK = _BW // _CH
_L = 16


def _tile_body(sexi_hbm, bmi_hbm, sex_hbm, bm_hbm, out_hbm, sexi_v, bmi_v,
               off_v, sex_t, bm_t, rows_a, rows_b, osems):
    wid = lax.axis_index("s") * _NC + lax.axis_index("c")
    base = wid * _BW
    pltpu.sync_copy(sexi_hbm.at[pl.ds(base, _BW)], sexi_v)
    pltpu.sync_copy(bmi_hbm.at[pl.ds(base, _BW)], bmi_v)
    pltpu.sync_copy(sex_hbm, sex_t)
    pltpu.sync_copy(bm_hbm, bm_t)
    for i in range(_BW // _L):
        sl = pl.ds(i * _L, _L)
        off_v[sl] = (sexi_v[sl] * _D) | ((bmi_v[sl] * _D) << 16)
    cps = [None, None]
    for j in range(_NCHUNK):
        buf = j % 2
        if cps[buf] is not None:
            cps[buf].wait()
        dst = (rows_a, rows_b)[buf]

        def group_body(g, _, j=j, dst=dst):
            ov = off_v[pl.ds(j * _CH + g * _L, _L)]
            offs = [ov[l] for l in range(_L)]
            for l in range(_L):
                so = offs[l] & 0xFFFF
                bo = lax.shift_right_logical(offs[l], 16)
                vs = ([sex_t[pl.ds(so + k, _L)] for k in range(0, _D, _L)]
                      + [bm_t[pl.ds(bo + k, _L)] for k in range(0, _D, _L)])
                for k, v in enumerate(vs):
                    dst[g * _L + l, pl.ds(k * _L, _L)] = v
            return 0

        lax.fori_loop(0, _CH // _L, group_body, 0)
        cps[buf] = pltpu.async_copy(
            dst,
            out_hbm.at[pl.ds(base + j * _CH, _CH)],
            osems.at[buf],
        )
    for cp in cps:
        cp.wait()


_lookup = pl.kernel(
    _tile_body,
    out_type=jax.ShapeDtypeStruct((_B, _DO), jnp.float32),
    mesh=plsc.VectorSubcoreMesh(core_axis_name="c", subcore_axis_name="s",
                                num_cores=_NC, num_subcores=_NS),
    compiler_params=pltpu.CompilerParams(needs_layout_passes=False),
    scratch_types=[
        pltpu.VMEM((_BW,), jnp.int32),
        pltpu.VMEM((_BW,), jnp.int32),
        pltpu.VMEM((_BW,), jnp.int32),
        pltpu.VMEM((3 * _D,), jnp.float32),
        pltpu.VMEM((13 * _D,), jnp.float32),
        pltpu.VMEM((_CH, _DO), jnp.float32),
        pltpu.VMEM((_CH, _DO), jnp.float32),
        pltpu.SemaphoreType.DMA((2,)),
    ],
)


def kernel(child_info, emb_sex, emb_b_month):
    info = child_info.astype(jnp.int32)
    return _lookup(info[:, 0], info[:, 1],
                   emb_sex.reshape(-1), emb_b_month.reshape(-1))
